# baseline (speedup 1.0000x reference)
import functools

import jax
import jax.numpy as jnp
from jax import lax
from jax.experimental import pallas as pl
from jax.experimental.pallas import tpu as pltpu
from jax.experimental.pallas import tpu_sc as plsc

BATCH = 16384
ROLE_DIM = 32
NUM_CORES = 2
NUM_SUBCORES = 16
NUM_WORKERS = NUM_CORES * NUM_SUBCORES
B_PER_W = BATCH // NUM_WORKERS
LANES = 16
GROUP = 8
NGROUPS = B_PER_W // GROUP


@functools.cache
def _build():
    mesh = plsc.VectorSubcoreMesh(core_axis_name="c", subcore_axis_name="s")

    @functools.partial(
        pl.kernel,
        mesh=mesh,
        out_type=jax.ShapeDtypeStruct((ROLE_DIM, BATCH), jnp.float32),
        scratch_types=[
            pltpu.VMEM((B_PER_W,), jnp.int32),
            pltpu.VMEM((GROUP, ROLE_DIM, 128), jnp.float32),
            pltpu.VMEM((GROUP, ROLE_DIM, 128), jnp.float32),
            pltpu.VMEM((ROLE_DIM, B_PER_W), jnp.float32),
            pltpu.SemaphoreType.DMA,
            pltpu.SemaphoreType.DMA,
        ],
        compiler_params=pltpu.CompilerParams(
            use_tc_tiling_on_sc=True, needs_layout_passes=False
        ),
    )
    def gather_kernel(
        idx_hbm, table_hbm, out_hbm, idx_v, buf_a, buf_b, blk_v, sem_a, sem_b
    ):
        wid = lax.axis_index("s") * NUM_CORES + lax.axis_index("c")
        base = wid * B_PER_W
        pltpu.sync_copy(idx_hbm.at[pl.ds(base, B_PER_W)], idx_v)

        lo = lax.iota(jnp.int32, LANES)
        hi = lo + LANES
        zero = jnp.zeros((LANES,), jnp.int32)

        def scalarize(g):
            vec = idx_v[pl.ds((g // 2) * LANES, LANES)]
            half = (g % 2) * GROUP
            return [
                jnp.max(jnp.where(lo == half + b, vec, zero))
                for b in range(GROUP)
            ]

        def issue(rs, buf, sem):
            cols = [pl.multiple_of((rs[b] >> 7) * 128, 128) for b in range(GROUP)]
            for p in range(4):
                for b in range(GROUP):
                    pltpu.make_async_copy(
                        table_hbm.at[pl.ds(8 * p, 8), pl.ds(cols[b], 128)],
                        buf.at[b, pl.ds(8 * p, 8), :],
                        sem,
                    ).start()

        def drain(rs, g, buf, sem):
            for _ in range(4 * GROUP):
                pltpu.make_async_copy(
                    table_hbm.at[pl.ds(0, 8), pl.ds(0, 128)],
                    buf.at[0, pl.ds(0, 8), :],
                    sem,
                ).wait()
            for b in range(GROUP):
                o = jnp.full((LANES,), rs[b] & 127, jnp.int32)
                col = jnp.full((LANES,), g * GROUP + b, jnp.int32)
                c0 = plsc.load_gather(buf.at[b], [lo, o])
                c1 = plsc.load_gather(buf.at[b], [hi, o])
                plsc.store_scatter(blk_v, [lo, col], c0)
                plsc.store_scatter(blk_v, [hi, col], c1)

        rs0 = scalarize(0)
        issue(rs0, buf_a, sem_a)

        def body(k, carry):
            g_a = 2 * k
            rs_b = scalarize(g_a + 1)
            issue(rs_b, buf_b, sem_b)
            rs_a = scalarize(g_a)
            drain(rs_a, g_a, buf_a, sem_a)

            @pl.when(k < NGROUPS // 2 - 1)
            def _issue_next():
                issue(scalarize(g_a + 2), buf_a, sem_a)

            drain(rs_b, g_a + 1, buf_b, sem_b)
            return carry

        lax.fori_loop(0, NGROUPS // 2, body, 0)
        pltpu.sync_copy(blk_v, out_hbm.at[:, pl.ds(base, B_PER_W)])

    return gather_kernel


def kernel(indices, embedding):
    out_t = _build()(indices, embedding.T)
    return out_t.T


# speedup vs baseline: 1.0036x; 1.0036x over previous
import functools

import jax
import jax.numpy as jnp
from jax import lax
from jax.experimental import pallas as pl
from jax.experimental.pallas import tpu as pltpu
from jax.experimental.pallas import tpu_sc as plsc

BATCH = 16384
ROLE_DIM = 32
NUM_CORES = 2
NUM_SUBCORES = 16
NUM_WORKERS = NUM_CORES * NUM_SUBCORES
B_PER_W = BATCH // NUM_WORKERS
LANES = 16
GROUP = 8
NGROUPS = B_PER_W // GROUP


@functools.cache
def _build():
    mesh = plsc.VectorSubcoreMesh(core_axis_name="c", subcore_axis_name="s")

    @functools.partial(
        pl.kernel,
        mesh=mesh,
        out_type=jax.ShapeDtypeStruct((ROLE_DIM, BATCH), jnp.float32),
        scratch_types=[
            pltpu.VMEM((B_PER_W,), jnp.int32),
            pltpu.VMEM((GROUP, ROLE_DIM, 128), jnp.float32),
            pltpu.VMEM((GROUP, ROLE_DIM, 128), jnp.float32),
            pltpu.VMEM((ROLE_DIM, B_PER_W), jnp.float32),
            pltpu.SemaphoreType.DMA,
            pltpu.SemaphoreType.DMA,
        ],
        compiler_params=pltpu.CompilerParams(
            use_tc_tiling_on_sc=True, needs_layout_passes=False
        ),
    )
    def gather_kernel(
        idx_hbm, table_hbm, out_hbm, idx_v, buf_a, buf_b, blk_v, sem_a, sem_b
    ):
        wid = lax.axis_index("s") * NUM_CORES + lax.axis_index("c")
        base = wid * B_PER_W
        pltpu.sync_copy(idx_hbm.at[pl.ds(base, B_PER_W)], idx_v)

        lo = lax.iota(jnp.int32, LANES)
        hi = lo + LANES
        zero = jnp.zeros((LANES,), jnp.int32)

        def scalarize(g):
            vec = idx_v[pl.ds((g // 2) * LANES, LANES)]
            half = (g % 2) * GROUP
            return [
                jnp.max(jnp.where(lo == half + b, vec, zero))
                for b in range(GROUP)
            ]

        def issue(rs, buf, sem):
            for b in range(GROUP):
                q = rs[b] >> 7
                pltpu.make_async_copy(
                    table_hbm.at[:, pl.ds(pl.multiple_of(q * 128, 128), 128)],
                    buf.at[b],
                    sem,
                ).start()

        def drain(rs, g, buf, sem):
            for b in range(GROUP):
                pltpu.make_async_copy(
                    table_hbm.at[:, pl.ds(0, 128)], buf.at[b], sem
                ).wait()
            for b in range(GROUP):
                o = jnp.full((LANES,), rs[b] & 127, jnp.int32)
                col = jnp.full((LANES,), g * GROUP + b, jnp.int32)
                c0 = plsc.load_gather(buf.at[b], [lo, o])
                c1 = plsc.load_gather(buf.at[b], [hi, o])
                plsc.store_scatter(blk_v, [lo, col], c0)
                plsc.store_scatter(blk_v, [hi, col], c1)

        rs0 = scalarize(0)
        issue(rs0, buf_a, sem_a)

        def body(k, carry):
            g_a = 2 * k
            rs_b = scalarize(g_a + 1)
            issue(rs_b, buf_b, sem_b)
            rs_a = scalarize(g_a)
            drain(rs_a, g_a, buf_a, sem_a)

            @pl.when(k < NGROUPS // 2 - 1)
            def _issue_next():
                issue(scalarize(g_a + 2), buf_a, sem_a)

            drain(rs_b, g_a + 1, buf_b, sem_b)
            return carry

        lax.fori_loop(0, NGROUPS // 2, body, 0)
        pltpu.sync_copy(blk_v, out_hbm.at[:, pl.ds(base, B_PER_W)])

    return gather_kernel


def kernel(indices, embedding):
    out_t = _build()(indices, embedding.T)
    return out_t.T


# speedup vs baseline: 1.0933x; 1.0893x over previous
import functools

import jax
import jax.numpy as jnp
from jax import lax
from jax.experimental import pallas as pl
from jax.experimental.pallas import tpu as pltpu
from jax.experimental.pallas import tpu_sc as plsc

BATCH = 16384
ROLE_DIM = 32
NUM_CORES = 2
NUM_SUBCORES = 16
NUM_WORKERS = NUM_CORES * NUM_SUBCORES
B_PER_W = BATCH // NUM_WORKERS
LANES = 16
GROUP = 8
NGROUPS = B_PER_W // GROUP


@functools.cache
def _build():
    mesh = plsc.VectorSubcoreMesh(core_axis_name="c", subcore_axis_name="s")

    @functools.partial(
        pl.kernel,
        mesh=mesh,
        out_type=jax.ShapeDtypeStruct((ROLE_DIM, BATCH), jnp.float32),
        scratch_types=[
            pltpu.VMEM((B_PER_W,), jnp.int32),
            pltpu.VMEM((GROUP, ROLE_DIM, 128), jnp.float32),
            pltpu.VMEM((GROUP, ROLE_DIM, 128), jnp.float32),
            pltpu.VMEM((GROUP, ROLE_DIM, 128), jnp.float32),
            pltpu.VMEM((ROLE_DIM, B_PER_W), jnp.float32),
            pltpu.SemaphoreType.DMA,
            pltpu.SemaphoreType.DMA,
            pltpu.SemaphoreType.DMA,
        ],
        compiler_params=pltpu.CompilerParams(
            use_tc_tiling_on_sc=True, needs_layout_passes=False
        ),
    )
    def gather_kernel(
        idx_hbm, table_hbm, out_hbm,
        idx_v, buf_a, buf_b, buf_c, blk_v, sem_a, sem_b, sem_c,
    ):
        wid = lax.axis_index("s") * NUM_CORES + lax.axis_index("c")
        base = wid * B_PER_W
        pltpu.sync_copy(idx_hbm.at[pl.ds(base, B_PER_W)], idx_v)

        lo = lax.iota(jnp.int32, LANES)
        hi = lo + LANES
        zero = jnp.zeros((LANES,), jnp.int32)

        def scalarize(g):
            vec = idx_v[pl.ds((g // 2) * LANES, LANES)]
            half = (g % 2) * GROUP
            return [
                jnp.max(jnp.where(lo == half + b, vec, zero))
                for b in range(GROUP)
            ]

        def issue(g, buf, sem):
            rs = scalarize(g)
            for b in range(GROUP):
                q = rs[b] >> 7
                pltpu.make_async_copy(
                    table_hbm.at[:, pl.ds(pl.multiple_of(q * 128, 128), 128)],
                    buf.at[b],
                    sem,
                ).start()

        def drain(g, buf, sem):
            rs = scalarize(g)
            for b in range(GROUP):
                pltpu.make_async_copy(
                    table_hbm.at[:, pl.ds(0, 128)], buf.at[b], sem
                ).wait()
            for b in range(GROUP):
                o = jnp.full((LANES,), rs[b] & 127, jnp.int32)
                col = jnp.full((LANES,), g * GROUP + b, jnp.int32)
                c0 = plsc.load_gather(buf.at[b], [lo, o])
                c1 = plsc.load_gather(buf.at[b], [hi, o])
                plsc.store_scatter(blk_v, [lo, col], c0)
                plsc.store_scatter(blk_v, [hi, col], c1)

        issue(0, buf_a, sem_a)
        issue(1, buf_b, sem_b)

        def body(j, carry):
            g = 3 * j
            issue(g + 2, buf_c, sem_c)
            drain(g, buf_a, sem_a)
            issue(g + 3, buf_a, sem_a)
            drain(g + 1, buf_b, sem_b)

            @pl.when(g + 4 < NGROUPS)
            def _issue_b():
                issue(g + 4, buf_b, sem_b)

            drain(g + 2, buf_c, sem_c)
            return carry

        lax.fori_loop(0, (NGROUPS - 1) // 3, body, 0)
        drain(NGROUPS - 1, buf_a, sem_a)
        pltpu.sync_copy(blk_v, out_hbm.at[:, pl.ds(base, B_PER_W)])

    return gather_kernel


def kernel(indices, embedding):
    out_t = _build()(indices, embedding.T)
    return out_t.T
